# baseline (speedup 1.0000x reference)
import functools

import jax
import jax.numpy as jnp
from jax.experimental import pallas as pl

IN_C = 3
CGEO = 256
N1 = 512
N2 = 128
RADII1 = (0.1, 0.2, 0.4)
NS1 = (16, 32, 128)
RADII2 = (0.2, 0.4, 0.8)
NS2 = (32, 64, 128)
K_FP = 3
B, N = 4, 4096


def _mlp(params, x, mask=None):
    for layer in params:
        x = x @ layer['W'].T
        if 'gamma' in layer:
            axes = tuple(range(x.ndim - 1))
            if mask is None:
                mean = jnp.mean(x, axis=axes)
                var = jnp.mean((x - mean) ** 2, axis=axes)
        else:
            x = x + layer['b']
            continue
        if mask is None:
            pass
        else:
            m = mask[..., None].astype(x.dtype)
            cnt = jnp.maximum(jnp.sum(m), 1.0)
            mean = jnp.sum(x * m, axis=axes) / cnt
            var = jnp.sum(((x - mean) ** 2) * m, axis=axes) / cnt
        x = layer['gamma'] * (x - mean) / jnp.sqrt(var + 1e-5) + layer['beta']
        x = jax.nn.relu(x)
    return x


def _fps_body(x_ref, y_ref, z_ref, px_ref, py_ref, pz_ref, *, n):
    X = x_ref[...]
    Y = y_ref[...]
    Z = z_ref[...]
    npts = X.shape[1]
    iota = jax.lax.broadcasted_iota(jnp.int32, X.shape, 1)
    iota_n = jax.lax.broadcasted_iota(jnp.int32, (X.shape[0], n), 1)
    cx0 = X[:, 0:1]
    cy0 = Y[:, 0:1]
    cz0 = Z[:, 0:1]
    accx0 = jnp.where(iota_n == 0, cx0, 0.0)
    accy0 = jnp.where(iota_n == 0, cy0, 0.0)
    accz0 = jnp.where(iota_n == 0, cz0, 0.0)
    d0 = jnp.full(X.shape, jnp.inf, jnp.float32)

    def body(i, carry):
        d, cx, cy, cz, ax, ay, az = carry
        dx = X - cx
        dy = Y - cy
        dz = Z - cz
        nd = (dx * dx + dy * dy) + dz * dz
        d = jnp.minimum(d, nd)
        rowmax = jnp.max(d, axis=1, keepdims=True)
        idx = jnp.min(jnp.where(d == rowmax, iota, npts), axis=1, keepdims=True)
        sel = iota == idx
        cx = jnp.sum(jnp.where(sel, X, 0.0), axis=1, keepdims=True)
        cy = jnp.sum(jnp.where(sel, Y, 0.0), axis=1, keepdims=True)
        cz = jnp.sum(jnp.where(sel, Z, 0.0), axis=1, keepdims=True)
        here = iota_n == i
        ax = jnp.where(here, cx, ax)
        ay = jnp.where(here, cy, ay)
        az = jnp.where(here, cz, az)
        return (d, cx, cy, cz, ax, ay, az)

    carry = (d0, cx0, cy0, cz0, accx0, accy0, accz0)
    carry = jax.lax.fori_loop(1, n, body, carry)
    _, _, _, _, ax, ay, az = carry
    px_ref[...] = ax
    py_ref[...] = ay
    pz_ref[...] = az


def _fps_pos(pos, n):
    b = pos.shape[0]
    X = pos[:, :, 0]
    Y = pos[:, :, 1]
    Z = pos[:, :, 2]
    px, py, pz = pl.pallas_call(
        functools.partial(_fps_body, n=n),
        out_shape=[jax.ShapeDtypeStruct((b, n), jnp.float32)] * 3,
    )(X, Y, Z)
    return jnp.stack([px, py, pz], axis=-1)


def _radius_neighbors(points, centers, r, k):
    d2 = jnp.sum((centers[:, :, None, :] - points[:, None, :, :]) ** 2, axis=-1)
    masked = jnp.where(d2 <= r * r, d2, jnp.inf)
    negv, idx = jax.lax.top_k(-masked, k)
    valid = jnp.isfinite(negv)
    return idx, valid


def _gather_b(x, idx):
    return jax.vmap(lambda a, i: a[i])(x, idx)


def _pointnet_conv(local_nn, x_src, pos_src, pos_dst, nbr_idx, valid):
    pos_j = _gather_b(pos_src, nbr_idx)
    rel = pos_j - pos_dst[:, :, None, :]
    x_j = _gather_b(x_src, nbr_idx)
    h = jnp.concatenate([x_j, rel], axis=-1)
    h = _mlp(local_nn, h, mask=valid)
    h = jnp.where(valid[..., None], h, -jnp.inf)
    out = jnp.max(h, axis=2)
    out = jnp.where(jnp.isfinite(out), out, 0.0)
    return out


def _multiscale_sa(convs, radii, ns, x, pos, pos_s):
    outs = []
    for r, k, p in zip(radii, ns, convs):
        nbr, valid = _radius_neighbors(pos, pos_s, r, k)
        outs.append(_pointnet_conv(p, x, pos, pos_s, nbr, valid))
    return jnp.concatenate(outs, axis=-1), pos_s


def _knn_body(posy_ref, posxt_ref, x_ref, o_ref, *, k):
    py = posy_ref[0]
    pxt = posxt_ref[0]
    xb = x_ref[0]
    m = py.shape[0]
    nx = pxt.shape[1]
    dx = py[:, 0:1] - pxt[0:1, :]
    dy = py[:, 1:2] - pxt[1:2, :]
    dz = py[:, 2:3] - pxt[2:3, :]
    d2 = (dx * dx + dy * dy) + dz * dz
    iota = jax.lax.broadcasted_iota(jnp.int32, d2.shape, 1)
    w = jnp.zeros_like(d2)
    s = jnp.zeros((m, 1), jnp.float32)
    d2w = d2
    for _ in range(k):
        v = jnp.min(d2w, axis=1, keepdims=True)
        i = jnp.min(jnp.where(d2w == v, iota, nx), axis=1, keepdims=True)
        here = iota == i
        wt = 1.0 / jnp.maximum(v, 1e-16)
        w = jnp.where(here, wt, w)
        s = s + wt
        d2w = jnp.where(here, jnp.inf, d2w)
    num = jnp.dot(w, xb, preferred_element_type=jnp.float32)
    o_ref[0] = num / s


def _knn_interpolate(x, pos_x, pos_y, k):
    b, ny, _ = pos_y.shape
    nx = pos_x.shape[1]
    c = x.shape[-1]
    m = min(ny, 512)
    pos_xt = jnp.transpose(pos_x, (0, 2, 1))
    return pl.pallas_call(
        functools.partial(_knn_body, k=k),
        grid=(b, ny // m),
        in_specs=[
            pl.BlockSpec((1, m, 3), lambda i, j: (i, j, 0)),
            pl.BlockSpec((1, 3, nx), lambda i, j: (i, 0, 0)),
            pl.BlockSpec((1, nx, c), lambda i, j: (i, 0, 0)),
        ],
        out_specs=pl.BlockSpec((1, m, c), lambda i, j: (i, j, 0)),
        out_shape=jax.ShapeDtypeStruct((b, ny, c), jnp.float32),
    )(pos_y, pos_xt, x)


def _copy_kernel(x_ref, o_ref):
    o_ref[...] = x_ref[...]


def _pl_identity(x):
    return pl.pallas_call(
        _copy_kernel,
        out_shape=jax.ShapeDtypeStruct(x.shape, x.dtype),
    )(x)


def kernel(pts, params):
    pos = pts
    x0 = pos
    pos1_s = _fps_pos(pos, N1)
    x1, pos1 = _multiscale_sa(params['sa1'], RADII1, NS1, x0, pos, pos1_s)
    pos2_s = _fps_pos(pos1, N2)
    x2, pos2 = _multiscale_sa(params['sa2'], RADII2, NS2, x1, pos1, pos2_s)
    g = _mlp(params['glob'], jnp.max(x2, axis=1))
    x1_up = _knn_interpolate(x2, pos2, pos1, K_FP)
    x1_fp = _mlp(params['fp1'], jnp.concatenate([x1_up, x1], axis=-1))
    x0_up = _knn_interpolate(x1_fp, pos1, pos, K_FP)
    F = _mlp(params['fp0'], jnp.concatenate([x0_up, x0], axis=-1))
    F = _pl_identity(F)
    return (F, g)
